# baseline (speedup 1.0000x reference)
import functools

import jax
import jax.numpy as jnp
from jax import lax
from jax.experimental import pallas as pl
from jax.experimental.pallas import tpu as pltpu
from jax.experimental.pallas import tpu_sc as plsc

NC = 2
NS = 16
NW = NC * NS

NBUF = 4
LOOKAHEAD = 3


@jax.jit
def kernel(token_ids, embedding_table):
    n_rows, row_len = token_ids.shape
    d = embedding_table.shape[1]
    ids = token_ids.astype(jnp.int32)
    table_p = jnp.pad(embedding_table, ((0, 0), (0, 128 - d)))
    assert n_rows % (NW * NBUF) == 0 and row_len % 8 == 0
    rows_per_w = n_rows // NW
    n_chunks = rows_per_w

    mesh = plsc.VectorSubcoreMesh(core_axis_name="c", subcore_axis_name="s")

    @functools.partial(
        pl.kernel,
        mesh=mesh,
        out_type=jax.ShapeDtypeStruct((n_rows, row_len, 128), jnp.float32),
        scratch_types=[
            pltpu.VMEM((rows_per_w, row_len), jnp.int32),
            pltpu.VMEM((NBUF, row_len, 128), jnp.float32),
            pltpu.SemaphoreType.DMA((NBUF,)),
            pltpu.SemaphoreType.DMA((NBUF,)),
        ],
        compiler_params=pltpu.CompilerParams(use_tc_tiling_on_sc=False),
    )
    def emb(table_hbm, idx_hbm, out_hbm, idx_v, rows_v, gsem, osem):
        wid = lax.axis_index("s") * NC + lax.axis_index("c")
        base = wid * rows_per_w
        pltpu.sync_copy(idx_hbm.at[pl.ds(base, rows_per_w), :], idx_v)

        def g_ref(c):
            return table_hbm.at[idx_v.at[c]]

        def o_ref(c):
            return out_hbm.at[base + c]

        def gather_start(c, bf):
            pltpu.async_copy(g_ref(c), rows_v.at[bf], gsem.at[bf])

        def gather_wait(c, bf):
            pltpu.make_async_copy(g_ref(c), rows_v.at[bf], gsem.at[bf]).wait()

        def out_start(c, bf):
            pltpu.async_copy(rows_v.at[bf], o_ref(c), osem.at[bf])

        def out_wait(c, bf):
            pltpu.make_async_copy(rows_v.at[bf], o_ref(c), osem.at[bf]).wait()

        for j in range(LOOKAHEAD):
            gather_start(j, j)

        gather_wait(0, 0)
        out_start(0, 0)
        gather_start(LOOKAHEAD, LOOKAHEAD % NBUF)
        for bf in range(1, NBUF):
            c = bf
            gather_wait(c, bf)
            out_start(c, bf)
            nf = (bf + LOOKAHEAD) % NBUF
            out_wait(c - 1, nf)
            gather_start(c + LOOKAHEAD, nf)

        @pl.loop(1, n_chunks // NBUF - 1)
        def _steady(i):
            c0 = i * NBUF
            for bf in range(NBUF):
                c = c0 + bf
                gather_wait(c, bf)
                out_start(c, bf)
                nf = (bf + LOOKAHEAD) % NBUF
                out_wait(c - 1, nf)
                gather_start(c + LOOKAHEAD, nf)

        c0 = n_chunks - NBUF
        gather_wait(c0, 0)
        out_start(c0, 0)
        out_wait(c0 - 1, LOOKAHEAD % NBUF)
        gather_start(c0 + LOOKAHEAD, LOOKAHEAD % NBUF)
        for bf in range(1, NBUF):
            c = c0 + bf
            gather_wait(c, bf)
            out_start(c, bf)
        for bf in range(NBUF):
            out_wait(c0 + bf, bf)

    out = emb(table_p, ids)
    return out[:, :, :d]


# speedup vs baseline: 1.0873x; 1.0873x over previous
import functools

import jax
import jax.numpy as jnp
from jax import lax
from jax.experimental import pallas as pl
from jax.experimental.pallas import tpu as pltpu
from jax.experimental.pallas import tpu_sc as plsc

NC = 2
NS = 16
NW = NC * NS

NBUF = 4
LOOKAHEAD = 3


@jax.jit
def kernel(token_ids, embedding_table):
    n_rows, row_len = token_ids.shape
    d = embedding_table.shape[1]
    ids = token_ids.astype(jnp.int32)
    assert n_rows % (NW * NBUF) == 0 and row_len % 8 == 0
    rows_per_w = n_rows // NW
    n_chunks = rows_per_w

    mesh = plsc.VectorSubcoreMesh(core_axis_name="c", subcore_axis_name="s")

    @functools.partial(
        pl.kernel,
        mesh=mesh,
        out_type=jax.ShapeDtypeStruct((n_rows, row_len, 128), jnp.float32),
        scratch_types=[
            pltpu.VMEM((rows_per_w, row_len), jnp.int32),
            pltpu.VMEM((NBUF, row_len, d), jnp.float32),
            pltpu.SemaphoreType.DMA((NBUF,)),
            pltpu.SemaphoreType.DMA((NBUF,)),
        ],
        compiler_params=pltpu.CompilerParams(use_tc_tiling_on_sc=False),
    )
    def emb(table_hbm, idx_hbm, out_hbm, idx_v, rows_v, gsem, osem):
        wid = lax.axis_index("s") * NC + lax.axis_index("c")
        base = wid * rows_per_w
        pltpu.sync_copy(idx_hbm.at[pl.ds(base, rows_per_w), :], idx_v)

        def g_ref(c):
            return table_hbm.at[idx_v.at[c]]

        def o_ref(c):
            return out_hbm.at[base + c, :, pl.ds(0, d)]

        def gather_start(c, bf):
            pltpu.async_copy(g_ref(c), rows_v.at[bf], gsem.at[bf])

        def gather_wait(c, bf):
            pltpu.make_async_copy(g_ref(c), rows_v.at[bf], gsem.at[bf]).wait()

        def out_start(c, bf):
            pltpu.async_copy(rows_v.at[bf], o_ref(c), osem.at[bf])

        def out_wait(c, bf):
            pltpu.make_async_copy(rows_v.at[bf], o_ref(c), osem.at[bf]).wait()

        for j in range(LOOKAHEAD):
            gather_start(j, j)

        gather_wait(0, 0)
        out_start(0, 0)
        gather_start(LOOKAHEAD, LOOKAHEAD % NBUF)
        for bf in range(1, NBUF):
            c = bf
            gather_wait(c, bf)
            out_start(c, bf)
            nf = (bf + LOOKAHEAD) % NBUF
            out_wait(c - 1, nf)
            gather_start(c + LOOKAHEAD, nf)

        @pl.loop(1, n_chunks // NBUF - 1)
        def _steady(i):
            c0 = i * NBUF
            for bf in range(NBUF):
                c = c0 + bf
                gather_wait(c, bf)
                out_start(c, bf)
                nf = (bf + LOOKAHEAD) % NBUF
                out_wait(c - 1, nf)
                gather_start(c + LOOKAHEAD, nf)

        c0 = n_chunks - NBUF
        gather_wait(c0, 0)
        out_start(c0, 0)
        out_wait(c0 - 1, LOOKAHEAD % NBUF)
        gather_start(c0 + LOOKAHEAD, LOOKAHEAD % NBUF)
        for bf in range(1, NBUF):
            c = c0 + bf
            gather_wait(c, bf)
            out_start(c, bf)
        for bf in range(NBUF):
            out_wait(c0 + bf, bf)

    out = emb(embedding_table, ids)
    return out[:, :, :d]
